# Initial kernel scaffold; baseline (speedup 1.0000x reference)
import jax
import jax.numpy as jnp
from jax.experimental import pallas as pl
from jax.experimental.pallas import tpu as pltpu

_H = 1000
_C = 128


def _copy_rows_body(table_ref, *rest):
    outs = rest[:_H]
    sem = rest[_H]
    copies = [
        pltpu.make_async_copy(table_ref.at[pl.ds(i, 1)], outs[i], sem)
        for i in range(_H)
    ]
    for c in copies:
        c.start()
    for c in copies:
        c.wait()


def kernel(table):
    outs = pl.pallas_call(
        _copy_rows_body,
        in_specs=[pl.BlockSpec(memory_space=pltpu.ANY)],
        out_specs=[pl.BlockSpec(memory_space=pltpu.ANY)] * _H,
        out_shape=[jax.ShapeDtypeStruct((1, _C), jnp.float32)] * _H,
        scratch_shapes=[pltpu.SemaphoreType.DMA],
    )(table)
    return {f"embeds_bb_{i}": {"codes": outs[i]} for i in range(_H)}


# speedup vs baseline: 3.5438x; 3.5438x over previous
import jax
import jax.numpy as jnp
from jax.experimental import pallas as pl
from jax.experimental.pallas import tpu as pltpu

_H = 1000
_C = 128


def _copy_rows_body(table_ref, *rest):
    outs = rest[:_H]
    sem = rest[_H]
    copies = [
        pltpu.make_async_copy(table_ref.at[pl.ds(i, 1)], outs[i], sem)
        for i in range(_H)
    ]
    for c in copies:
        c.start()
    for c in copies:
        c.wait()


def kernel(table):
    outs = pl.pallas_call(
        _copy_rows_body,
        in_specs=[pl.BlockSpec(memory_space=pl.ANY)],
        out_specs=[pl.BlockSpec(memory_space=pl.ANY)] * _H,
        out_shape=[jax.ShapeDtypeStruct((1, _C), jnp.float32)] * _H,
        scratch_shapes=[pltpu.SemaphoreType.DMA],
    )(table)
    return {f"embeds_bb_{i}": {"codes": outs[i]} for i in range(_H)}
